# baseline (speedup 1.0000x reference)
import jax
import jax.numpy as jnp
from jax.experimental import pallas as pl
from jax.experimental.pallas import tpu as pltpu

_TOP_K = 2
_TILE = 2048


def _top2(logits):
    m1 = jnp.max(logits, axis=1)
    i1 = jnp.argmax(logits, axis=1).astype(jnp.int32)
    col = jax.lax.broadcasted_iota(jnp.int32, logits.shape, 1)
    masked = jnp.where(col == i1[:, None], -jnp.inf, logits)
    m2 = jnp.max(masked, axis=1)
    i2 = jnp.argmax(masked, axis=1).astype(jnp.int32)
    t = jnp.exp(m2 - m1)
    g1 = 1.0 / (1.0 + t)
    g2 = t / (1.0 + t)
    idx = jnp.stack([i1, i2], axis=1)
    gates = jnp.stack([g1, g2], axis=1)
    return idx, gates


def _router_kernel(xa_ref, xb_ref, w_ref, idxa_ref, gatea_ref,
                   idxb_ref, gateb_ref):
    w = w_ref[...]
    dims = (((1,), (1,)), ((), ()))
    la = jax.lax.dot_general(xa_ref[...], w, dims,
                             preferred_element_type=jnp.float32)
    idxa_ref[...], gatea_ref[...] = _top2(la)
    lb = jax.lax.dot_general(xb_ref[...], w, dims,
                             preferred_element_type=jnp.float32)
    idxb_ref[...], gateb_ref[...] = _top2(lb)


@jax.jit
def kernel(x, W):
    n, d = x.shape
    e = W.shape[0]
    g = n // (2 * _TILE)
    ia, ga, ib, gb = pl.pallas_call(
        _router_kernel,
        grid=(g,),
        in_specs=[
            pl.BlockSpec((_TILE, d), lambda i: (2 * i, 0)),
            pl.BlockSpec((_TILE, d), lambda i: (2 * i + 1, 0)),
            pl.BlockSpec((e, d), lambda i: (0, 0)),
        ],
        out_specs=[
            pl.BlockSpec((_TILE, _TOP_K), lambda i: (i, 0)),
            pl.BlockSpec((_TILE, _TOP_K), lambda i: (i, 0)),
            pl.BlockSpec((_TILE, _TOP_K), lambda i: (i, 0)),
            pl.BlockSpec((_TILE, _TOP_K), lambda i: (i, 0)),
        ],
        out_shape=[
            jax.ShapeDtypeStruct((n // 2, _TOP_K), jnp.int32),
            jax.ShapeDtypeStruct((n // 2, _TOP_K), jnp.float32),
            jax.ShapeDtypeStruct((n // 2, _TOP_K), jnp.int32),
            jax.ShapeDtypeStruct((n // 2, _TOP_K), jnp.float32),
        ],
        compiler_params=pltpu.CompilerParams(
            dimension_semantics=("parallel",)),
    )(x, x, W)
    idx = jnp.concatenate(
        [ia.reshape(g, _TILE, _TOP_K), ib.reshape(g, _TILE, _TOP_K)],
        axis=1).reshape(n, _TOP_K)
    gates = jnp.concatenate(
        [ga.reshape(g, _TILE, _TOP_K), gb.reshape(g, _TILE, _TOP_K)],
        axis=1).reshape(n, _TOP_K)
    return idx, gates


# speedup vs baseline: 1.0804x; 1.0804x over previous
import jax
import jax.numpy as jnp
from jax.experimental import pallas as pl
from jax.experimental.pallas import tpu as pltpu

_TOP_K = 2
_TILE = 4096


def _router_kernel(x_ref, w_ref, idx_ref, gate_ref):
    x = x_ref[...]
    w = w_ref[...]
    logits = jax.lax.dot_general(
        x, w, (((1,), (1,)), ((), ())),
        preferred_element_type=jnp.float32)

    m1 = jnp.max(logits, axis=1)
    i1 = jnp.argmax(logits, axis=1).astype(jnp.int32)
    col = jax.lax.broadcasted_iota(jnp.int32, logits.shape, 1)
    masked = jnp.where(col == i1[:, None], -jnp.inf, logits)
    m2 = jnp.max(masked, axis=1)
    i2 = jnp.argmax(masked, axis=1).astype(jnp.int32)

    t = jnp.exp(m2 - m1)
    g1 = 1.0 / (1.0 + t)
    g2 = t / (1.0 + t)

    idx_ref[...] = jnp.stack([i1, i2], axis=1)
    gate_ref[...] = jnp.stack([g1, g2], axis=1)


@jax.jit
def kernel(x, W):
    n, d = x.shape
    e = W.shape[0]
    grid = (n // _TILE,)
    idx, gates = pl.pallas_call(
        _router_kernel,
        grid=grid,
        in_specs=[
            pl.BlockSpec((_TILE, d), lambda i: (i, 0)),
            pl.BlockSpec((e, d), lambda i: (0, 0)),
        ],
        out_specs=[
            pl.BlockSpec((_TILE, _TOP_K), lambda i: (i, 0)),
            pl.BlockSpec((_TILE, _TOP_K), lambda i: (i, 0)),
        ],
        out_shape=[
            jax.ShapeDtypeStruct((n, _TOP_K), jnp.int32),
            jax.ShapeDtypeStruct((n, _TOP_K), jnp.float32),
        ],
        compiler_params=pltpu.CompilerParams(
            dimension_semantics=("parallel",)),
    )(x, W)
    return idx, gates
